# baseline (speedup 1.0000x reference)
import functools

import jax
import jax.numpy as jnp
from jax import lax
from jax.experimental import pallas as pl
from jax.experimental.pallas import tpu as pltpu
from jax.experimental.pallas import tpu_sc as plsc

_B, _S, _H = 4, 2048, 128
_N = _B * _S
_EPS = 1e-5
_NC, _NS = 2, 16
_NW = _NC * _NS
_WPB = _NW // _B
_TPW = _S // _WPB
_CHUNK = 128
_NCH = _TPW // _CHUNK


@functools.cache
def _gather_sum_kernel():
    mesh = plsc.VectorSubcoreMesh(core_axis_name="c", subcore_axis_name="s",
                                  num_cores=_NC, num_subcores=_NS)

    @functools.partial(
        pl.kernel,
        out_type=jax.ShapeDtypeStruct((_N, _H), jnp.float32),
        mesh=mesh,
        scratch_types=[
            pltpu.VMEM((_TPW,), jnp.int32),
            pltpu.VMEM((_TPW,), jnp.int32),
            pltpu.VMEM((_TPW, _H), jnp.float32),
            pltpu.SemaphoreType.DMA,
        ],
    )
    def body(ids_hbm, tt_hbm, word_hbm, seg_hbm, out_hbm,
             idx_v, tt_v, rows_v, sem):
        wid = lax.axis_index("s") * _NC + lax.axis_index("c")
        b = wid // _WPB
        col0 = (wid % _WPB) * _TPW

        pltpu.sync_copy(ids_hbm.at[b, pl.ds(col0, _TPW)], idx_v)
        pltpu.sync_copy(tt_hbm.at[b, pl.ds(col0, _TPW)], tt_v)

        seg_cps = [
            pltpu.async_copy(
                seg_hbm.at[tt_v.at[pl.ds(j * _CHUNK, _CHUNK)]],
                rows_v.at[pl.ds(j * _CHUNK, _CHUNK)],
                sem,
            )
            for j in range(_NCH)
        ]
        for c in seg_cps:
            c.wait()
        word_cps = [
            pltpu.async_copy(
                word_hbm.at[idx_v.at[pl.ds(j * _CHUNK, _CHUNK)]],
                rows_v.at[pl.ds(j * _CHUNK, _CHUNK)],
                sem,
                add=True,
            )
            for j in range(_NCH)
        ]
        for c in word_cps:
            c.wait()

        pltpu.sync_copy(rows_v, out_hbm.at[pl.ds(wid * _TPW, _TPW)])

    return body


_BLK = 512
_PBLK = _S // _BLK


def _pos_ln_body(x_ref, pos_ref, gam_ref, bet_ref, o_ref):
    i = pl.program_id(0)
    off = lax.rem(i, _PBLK) * _BLK
    x = x_ref[...] + pos_ref[pl.ds(off, _BLK), :]
    mean = jnp.mean(x, axis=-1, keepdims=True)
    xc = x - mean
    var = jnp.mean(xc * xc, axis=-1, keepdims=True)
    o_ref[...] = xc * lax.rsqrt(var + _EPS) * gam_ref[...] + bet_ref[...]


def _pos_ln(summed, pos_emb, gamma, beta):
    return pl.pallas_call(
        _pos_ln_body,
        grid=(_N // _BLK,),
        in_specs=[
            pl.BlockSpec((_BLK, _H), lambda i: (i, 0)),
            pl.BlockSpec((_S, _H), lambda i: (0, 0)),
            pl.BlockSpec((1, _H), lambda i: (0, 0)),
            pl.BlockSpec((1, _H), lambda i: (0, 0)),
        ],
        out_specs=pl.BlockSpec((_BLK, _H), lambda i: (i, 0)),
        out_shape=jax.ShapeDtypeStruct((_N, _H), jnp.float32),
    )(summed, pos_emb, gamma, beta)


def kernel(input_ids, token_type_ids, word_emb, pos_emb, seg_emb, gamma, beta):
    ids = input_ids.astype(jnp.int32)
    tt = token_type_ids.astype(jnp.int32)
    summed = _gather_sum_kernel()(ids, tt, word_emb, seg_emb)
    out = _pos_ln(summed, pos_emb, gamma.reshape(1, _H), beta.reshape(1, _H))
    return out.reshape(_B, _S, _H)


# speedup vs baseline: 6.2990x; 6.2990x over previous
import functools

import jax
import jax.numpy as jnp
from jax import lax
from jax.experimental import pallas as pl
from jax.experimental.pallas import tpu as pltpu
from jax.experimental.pallas import tpu_sc as plsc

_B, _S, _H = 4, 2048, 128
_N = _B * _S
_EPS = 1e-5
_NC, _NS = 2, 16
_NW = _NC * _NS
_WPB = _NW // _B
_TPW = _S // _WPB
_CHUNK = 128
_NCH = _TPW // _CHUNK


@functools.cache
def _gather_words_kernel():
    mesh = plsc.VectorSubcoreMesh(core_axis_name="c", subcore_axis_name="s",
                                  num_cores=_NC, num_subcores=_NS)

    @functools.partial(
        pl.kernel,
        out_type=jax.ShapeDtypeStruct((_N, _H), jnp.float32),
        mesh=mesh,
        scratch_types=[
            pltpu.VMEM((_TPW,), jnp.int32),
            pltpu.VMEM((_TPW, _H), jnp.float32),
            pltpu.SemaphoreType.DMA,
        ],
    )
    def body(ids_hbm, word_hbm, out_hbm, idx_v, rows_v, sem):
        wid = lax.axis_index("s") * _NC + lax.axis_index("c")
        b = wid // _WPB
        col0 = (wid % _WPB) * _TPW

        pltpu.sync_copy(ids_hbm.at[b, pl.ds(col0, _TPW)], idx_v)
        copies = [
            pltpu.async_copy(
                word_hbm.at[idx_v.at[pl.ds(j * _CHUNK, _CHUNK)]],
                rows_v.at[pl.ds(j * _CHUNK, _CHUNK)],
                sem,
            )
            for j in range(_NCH)
        ]
        for c in copies:
            c.wait()
        pltpu.sync_copy(rows_v, out_hbm.at[pl.ds(wid * _TPW, _TPW)])

    return body


_BLK = 4096
_PBLK = max(_S // _BLK, 1)


def _add_ln_body(x_ref, pos_ref, ttf_ref, seg_ref, gam_ref, bet_ref, o_ref):
    s0 = seg_ref[0:1, :]
    dseg = seg_ref[1:2, :] - s0
    x = x_ref[...].reshape(_BLK // _S, _S, _H) + pos_ref[...][None]
    x = x.reshape(_BLK, _H) + s0 + ttf_ref[...] * dseg
    mean = jnp.mean(x, axis=-1, keepdims=True)
    xc = x - mean
    var = jnp.mean(xc * xc, axis=-1, keepdims=True)
    o_ref[...] = xc * lax.rsqrt(var + _EPS) * gam_ref[...] + bet_ref[...]


def _add_ln(gathered, pos_emb, ttf, seg_emb, gamma, beta):
    return pl.pallas_call(
        _add_ln_body,
        grid=(_N // _BLK,),
        in_specs=[
            pl.BlockSpec((_BLK, _H), lambda i: (i, 0)),
            pl.BlockSpec((_S, _H), lambda i: (0, 0)),
            pl.BlockSpec((_BLK, 1), lambda i: (i, 0)),
            pl.BlockSpec((2, _H), lambda i: (0, 0)),
            pl.BlockSpec((1, _H), lambda i: (0, 0)),
            pl.BlockSpec((1, _H), lambda i: (0, 0)),
        ],
        out_specs=pl.BlockSpec((_BLK, _H), lambda i: (i, 0)),
        out_shape=jax.ShapeDtypeStruct((_N, _H), jnp.float32),
    )(gathered, pos_emb, ttf, seg_emb, gamma, beta)


def kernel(input_ids, token_type_ids, word_emb, pos_emb, seg_emb, gamma, beta):
    ids = input_ids.astype(jnp.int32)
    gathered = _gather_words_kernel()(ids, word_emb)
    ttf = token_type_ids.astype(jnp.float32).reshape(_N, 1)
    out = _add_ln(gathered, pos_emb, ttf, seg_emb,
                  gamma.reshape(1, _H), beta.reshape(1, _H))
    return out.reshape(_B, _S, _H)
